# Initial kernel scaffold; baseline (speedup 1.0000x reference)
import jax
import jax.numpy as jnp
from jax.experimental import pallas as pl


def kernel(x, edge_index, edge_attr, batch, W, a_src, a_dst, W_e, a_e, b, W_lin, b_lin):
    raise NotImplementedError("write your pallas kernel here")


# speedup vs baseline: 11.1656x; 11.1656x over previous
import functools
import jax
import jax.numpy as jnp
from jax import lax
from jax.experimental import pallas as pl
from jax.experimental.pallas import tpu as pltpu
from jax.experimental.pallas import tpu_sc as plsc

_NS = 16
_N = 10000
_NP = 10240
_E = 320000
_CH = 157
_EPT = _CH * 128
_NEG = -1e30
_HALF = _NP // 2
_ACC = _HALF + 128


def _leaky(t):
    return jnp.maximum(t, 0.2 * t)


def _bcast_lane(v, r):
    idx = jnp.full((16, 1), r, dtype=jnp.int32)
    dn = lax.GatherDimensionNumbers(
        offset_dims=(), collapsed_slice_dims=(0,), start_index_map=(0,))
    return lax.gather(v, idx, dn, (1,),
                      mode=lax.GatherScatterMode.PROMISE_IN_BOUNDS)


@functools.partial(
    pl.kernel,
    out_type=[
        jax.ShapeDtypeStruct((_NP, 128), jnp.float32),
        jax.ShapeDtypeStruct((_NP,), jnp.float32),
    ],
    mesh=plsc.VectorSubcoreMesh(core_axis_name="c", subcore_axis_name="s"),
    compiler_params=pltpu.CompilerParams(needs_layout_passes=False),
    scratch_types=[
        pltpu.VMEM((128,), jnp.int32),
        pltpu.VMEM((128,), jnp.int32),
        pltpu.VMEM((128,), jnp.float32),
        pltpu.VMEM((128,), jnp.float32),
        pltpu.VMEM((_N,), jnp.float32),
        pltpu.VMEM((_N,), jnp.float32),
        pltpu.VMEM((128, 128), jnp.float32),
        pltpu.VMEM((_ACC // _NS,), jnp.float32),
        pltpu.VMEM((16,), jnp.float32),
        pltpu.SemaphoreType.DMA,
        pltpu.VMEM_SHARED((_ACC,), jnp.float32),
        pltpu.VMEM_SHARED((_ACC, 128), jnp.float32),
    ],
)
def _sc_edge(src_hbm, dst_hbm, ae_hbm, as_hbm, ad_hbm, h_hbm, m_hbm,
             u_hbm, d_hbm,
             src_c, dst_c, ae_c, ex_c, as_v, ad_v, rows_v, zb_v, m_v, sem,
             denom_sh, out_sh):
    cid = lax.axis_index("c")
    sid = lax.axis_index("s")
    node0 = cid * _HALF

    pltpu.sync_copy(as_hbm, as_v)
    pltpu.sync_copy(ad_hbm, ad_v)
    pltpu.sync_copy(m_hbm, m_v)

    zv = jnp.zeros((16,), jnp.float32)

    def _zrow(i, _):
        for c in range(8):
            rows_v[i, pl.ds(c * 16, 16)] = zv
        return 0
    lax.fori_loop(0, 128, _zrow, 0)

    apt = _ACC // _NS

    def _zb16(i, _):
        zb_v[pl.ds(i * 16, 16)] = zv
        return 0
    lax.fori_loop(0, apt // 16, _zb16, 0)
    zb_v[pl.ds(apt - 16, 16)] = zv

    row0 = pl.multiple_of(sid * apt, 8)
    pltpu.sync_copy(rows_v, out_sh.at[pl.ds(row0, 128)])
    pltpu.sync_copy(rows_v, out_sh.at[pl.ds(row0 + 128, 128)])
    pltpu.sync_copy(rows_v.at[pl.ds(0, apt - 256)],
                    out_sh.at[pl.ds(row0 + 256, apt - 256)])
    pltpu.sync_copy(zb_v, denom_sh.at[pl.ds(row0, apt)])

    plsc.subcore_barrier()

    m16 = m_v[...]

    def _chunk(j, _):
        pltpu.sync_copy(src_hbm.at[sid, j], src_c)
        pltpu.sync_copy(dst_hbm.at[sid, j], dst_c)
        pltpu.sync_copy(ae_hbm.at[sid, j], ae_c)
        gat = pltpu.async_copy(h_hbm.at[src_c], rows_v, sem)
        for k in range(8):
            si = src_c[pl.ds(k * 16, 16)]
            di = dst_c[pl.ds(k * 16, 16)]
            a16 = ae_c[pl.ds(k * 16, 16)]
            sg = plsc.load_gather(as_v, [si])
            dg = plsc.load_gather(ad_v, [di])
            t = sg + dg + a16
            ex_c[pl.ds(k * 16, 16)] = jnp.exp(_leaky(t) - m16)
            ld = di - node0
            oor = (ld < 0) | (ld >= _HALF)
            trash = _HALF + (di & 127)
            dst_c[pl.ds(k * 16, 16)] = jnp.where(oor, trash, ld)
        gat.wait()
        for k in range(8):
            e16 = ex_c[pl.ds(k * 16, 16)]
            for r in range(16):
                b16 = _bcast_lane(e16, r)
                row = k * 16 + r
                for c in range(8):
                    rows_v[row, pl.ds(c * 16, 16)] = (
                        rows_v[row, pl.ds(c * 16, 16)] * b16)
        pltpu.sync_copy(ex_c, denom_sh.at[dst_c], add=True)
        pltpu.sync_copy(rows_v, out_sh.at[dst_c], add=True)
        return 0
    lax.fori_loop(0, _CH, _chunk, 0)

    plsc.subcore_barrier()

    rpt = _HALF // _NS
    out0 = pl.multiple_of(sid * rpt, 8)
    h0 = pl.multiple_of(cid * _HALF + sid * rpt, 8)
    pltpu.sync_copy(out_sh.at[pl.ds(out0, rpt)], u_hbm.at[pl.ds(h0, rpt)])
    pltpu.sync_copy(denom_sh.at[pl.ds(out0, rpt)], zb_v.at[pl.ds(0, rpt)])
    pltpu.sync_copy(zb_v.at[pl.ds(0, rpt)], d_hbm.at[pl.ds(h0, rpt)])


_BLK = 1024
_GRID = _NP // _BLK


def _store_h_sd_max(h, h_ref, sd_ref, mx_ref, a2_ref):
    sd = jnp.dot(h, a2_ref[...], preferred_element_type=jnp.float32)
    h_ref[...] = h
    sd_ref[...] = sd
    i = pl.program_id(0)
    rid = i * _BLK + lax.broadcasted_iota(jnp.int32, (_BLK, 8), 0)
    mb = jnp.broadcast_to(
        jnp.max(jnp.where(rid < _N, sd, _NEG), axis=0, keepdims=True), (8, 8))

    @pl.when(i == 0)
    def _():
        mx_ref[...] = mb

    @pl.when(i > 0)
    def _():
        mx_ref[...] = jnp.maximum(mx_ref[...], mb)


def _mm_first_body(x_ref, w_ref, a2_ref, h_ref, sd_ref, mx_ref):
    h = jnp.dot(x_ref[...], w_ref[...], preferred_element_type=jnp.float32)
    _store_h_sd_max(h, h_ref, sd_ref, mx_ref, a2_ref)


def _mm_later_body(u_ref, d_ref, bias_ref, w_ref, a2_ref,
                   h_ref, sd_ref, mx_ref):
    den = d_ref[...] + 1e-16
    y = u_ref[...] / den + bias_ref[...]
    y = jnp.maximum(y, 0.0)
    h = jnp.dot(y, w_ref[...], preferred_element_type=jnp.float32)
    _store_h_sd_max(h, h_ref, sd_ref, mx_ref, a2_ref)


_MM_OUT = [
    jax.ShapeDtypeStruct((_NP, 128), jnp.float32),
    jax.ShapeDtypeStruct((_NP, 8), jnp.float32),
    jax.ShapeDtypeStruct((8, 8), jnp.float32),
]
_MM_OUT_SPECS = [
    pl.BlockSpec((_BLK, 128), lambda i: (i, 0)),
    pl.BlockSpec((_BLK, 8), lambda i: (i, 0)),
    pl.BlockSpec((8, 8), lambda i: (0, 0)),
]

_mm_first = pl.pallas_call(
    _mm_first_body,
    grid=(_GRID,),
    in_specs=[
        pl.BlockSpec((_BLK, 128), lambda i: (i, 0)),
        pl.BlockSpec((128, 128), lambda i: (0, 0)),
        pl.BlockSpec((128, 8), lambda i: (0, 0)),
    ],
    out_specs=_MM_OUT_SPECS,
    out_shape=_MM_OUT,
)

_mm_later = pl.pallas_call(
    _mm_later_body,
    grid=(_GRID,),
    in_specs=[
        pl.BlockSpec((_BLK, 128), lambda i: (i, 0)),
        pl.BlockSpec((_BLK, 1), lambda i: (i, 0)),
        pl.BlockSpec((1, 128), lambda i: (0, 0)),
        pl.BlockSpec((128, 128), lambda i: (0, 0)),
        pl.BlockSpec((128, 8), lambda i: (0, 0)),
    ],
    out_specs=_MM_OUT_SPECS,
    out_shape=_MM_OUT,
)


def _ae_body(ea_ref, we_ref, aew_ref, out_ref, mx_ref):
    we = we_ref[...]
    aw = aew_ref[...]
    cols = [jnp.dot(we[l], aw[l], preferred_element_type=jnp.float32)[:, None]
            for l in range(5)]
    cols.append(jnp.zeros((12, 3), jnp.float32))
    v8 = jnp.concatenate(cols, axis=1)
    ae8 = jnp.dot(ea_ref[...], v8, preferred_element_type=jnp.float32)
    out_ref[...] = ae8
    i = pl.program_id(0)
    mb = jnp.broadcast_to(jnp.max(ae8, axis=0, keepdims=True), (8, 8))

    @pl.when(i == 0)
    def _():
        mx_ref[...] = mb

    @pl.when(i > 0)
    def _():
        mx_ref[...] = jnp.maximum(mx_ref[...], mb)


_AE_BLK = 2000
_ae_proj = pl.pallas_call(
    _ae_body,
    grid=(_E // _AE_BLK,),
    in_specs=[
        pl.BlockSpec((_AE_BLK, 12), lambda i: (i, 0)),
        pl.BlockSpec((5, 12, 128), lambda i: (0, 0, 0)),
        pl.BlockSpec((5, 128), lambda i: (0, 0)),
    ],
    out_specs=[
        pl.BlockSpec((_AE_BLK, 8), lambda i: (i, 0)),
        pl.BlockSpec((8, 8), lambda i: (0, 0)),
    ],
    out_shape=[
        jax.ShapeDtypeStruct((_E, 8), jnp.float32),
        jax.ShapeDtypeStruct((8, 8), jnp.float32),
    ],
)


def _pool_body(u_ref, d_ref, bias_ref, bf_ref, wl_ref, bl_ref,
               out_ref, pooled_acc, counts_acc):
    i = pl.program_id(0)

    @pl.when(i == 0)
    def _():
        pooled_acc[...] = jnp.zeros((64, 128), jnp.float32)
        counts_acc[...] = jnp.zeros((64, 128), jnp.float32)

    den = d_ref[...] + 1e-16
    y = u_ref[...] / den + bias_ref[...]
    bf = bf_ref[...]
    g = lax.broadcasted_iota(jnp.int32, (_BLK, 64), 1)
    mask = (bf == g).astype(jnp.float32)
    pooled_acc[...] += lax.dot_general(
        mask, y, (((0,), (0,)), ((), ())), preferred_element_type=jnp.float32)
    counts_acc[...] += jnp.broadcast_to(
        jnp.sum(mask, axis=0)[:, None], (64, 128))

    @pl.when(i == _GRID - 1)
    def _():
        pm = pooled_acc[...] / jnp.maximum(counts_acc[...], 1.0)
        out_ref[...] = jnp.dot(pm, wl_ref[...],
                               preferred_element_type=jnp.float32) + bl_ref[...]


_pool = pl.pallas_call(
    _pool_body,
    grid=(_GRID,),
    in_specs=[
        pl.BlockSpec((_BLK, 128), lambda i: (i, 0)),
        pl.BlockSpec((_BLK, 1), lambda i: (i, 0)),
        pl.BlockSpec((1, 128), lambda i: (0, 0)),
        pl.BlockSpec((_BLK, 1), lambda i: (i, 0)),
        pl.BlockSpec((128, 1), lambda i: (0, 0)),
        pl.BlockSpec((1, 1), lambda i: (0, 0)),
    ],
    out_specs=pl.BlockSpec((64, 1), lambda i: (0, 0)),
    out_shape=jax.ShapeDtypeStruct((64, 1), jnp.float32),
    scratch_shapes=[
        pltpu.VMEM((64, 128), jnp.float32),
        pltpu.VMEM((64, 128), jnp.float32),
    ],
)


def _tile_layout(a, pad_val):
    a2 = a.reshape(_NS, _E // _NS)
    pad = jnp.full((_NS, _EPT - _E // _NS), pad_val, a.dtype)
    return jnp.concatenate([a2, pad], axis=1).reshape(_NS, _CH, 128)


def kernel(x, edge_index, edge_attr, batch, W, a_src, a_dst, W_e, a_e, b,
           W_lin, b_lin):
    f32 = jnp.float32
    src_t = _tile_layout(edge_index[0], 0)
    dst_t = _tile_layout(edge_index[1], 0)

    ae8, mxae = _ae_proj(edge_attr, W_e, a_e)

    x_pad = jnp.concatenate([x, jnp.zeros((_NP - _N, 128), f32)], axis=0)
    batchf = jnp.concatenate(
        [batch, jnp.full((_NP - _N,), 64, jnp.int32)])[:, None]

    u = d = None
    for l in range(5):
        a2 = jnp.concatenate(
            [a_src[l][:, None], a_dst[l][:, None], jnp.zeros((128, 6), f32)],
            axis=1)
        if l == 0:
            h, sd, mxsd = _mm_first(x_pad, W[0], a2)
        else:
            h, sd, mxsd = _mm_later(u, d[:, None], b[l - 1][None, :],
                                    W[l], a2)
        as_l = sd[:_N, 0]
        ad_l = sd[:_N, 1]
        m_val = _leaky(mxsd[0, 0] + mxsd[0, 1] + mxae[0, l])
        m_arr = jnp.full((16,), m_val, f32)
        ae_l = _tile_layout(ae8[:, l], _NEG)
        u, d = _sc_edge(src_t, dst_t, ae_l, as_l, ad_l, h, m_arr)

    return _pool(u, d[:, None], b[4][None, :], batchf, W_lin,
                 b_lin[:, None])


# speedup vs baseline: 11.6500x; 1.0434x over previous
import functools
import jax
import jax.numpy as jnp
from jax import lax
from jax.experimental import pallas as pl
from jax.experimental.pallas import tpu as pltpu
from jax.experimental.pallas import tpu_sc as plsc

_NS = 16
_N = 10000
_NP = 10240
_E = 320000
_CH = 158
_EPT = _CH * 128
_NEG = -1e30
_HALF = _NP // 2
_ACC = _HALF + 128


def _leaky(t):
    return jnp.maximum(t, 0.2 * t)


def _bcast_lane(v, r):
    idx = jnp.full((16, 1), r, dtype=jnp.int32)
    dn = lax.GatherDimensionNumbers(
        offset_dims=(), collapsed_slice_dims=(0,), start_index_map=(0,))
    return lax.gather(v, idx, dn, (1,),
                      mode=lax.GatherScatterMode.PROMISE_IN_BOUNDS)


@functools.partial(
    pl.kernel,
    out_type=[
        jax.ShapeDtypeStruct((_NP, 128), jnp.float32),
        jax.ShapeDtypeStruct((_NP,), jnp.float32),
    ],
    mesh=plsc.VectorSubcoreMesh(core_axis_name="c", subcore_axis_name="s"),
    compiler_params=pltpu.CompilerParams(needs_layout_passes=False),
    scratch_types=[
        pltpu.VMEM((2, 384), jnp.int32),
        pltpu.VMEM((2, 128), jnp.int32),
        pltpu.VMEM((2, 128), jnp.float32),
        pltpu.VMEM((_N,), jnp.float32),
        pltpu.VMEM((_N,), jnp.float32),
        pltpu.VMEM((2, 128, 128), jnp.float32),
        pltpu.VMEM((_ACC // _NS,), jnp.float32),
        pltpu.VMEM((16,), jnp.float32),
        pltpu.SemaphoreType.DMA,
        pltpu.SemaphoreType.DMA,
        pltpu.SemaphoreType.DMA,
        pltpu.SemaphoreType.DMA,
        pltpu.SemaphoreType.DMA,
        pltpu.SemaphoreType.DMA,
        pltpu.VMEM_SHARED((_ACC,), jnp.float32),
        pltpu.VMEM_SHARED((_ACC, 128), jnp.float32),
    ],
)
def _sc_edge(edges_hbm, as_hbm, ad_hbm, h_hbm, m_hbm,
             u_hbm, d_hbm,
             inb, ldst, exb, as_v, ad_v, rows_v, zb_v, m_v,
             sg0, sg1, ss0, ss1, sd0, sd1,
             denom_sh, out_sh):
    cid = lax.axis_index("c")
    sid = lax.axis_index("s")
    node0 = cid * _HALF
    sg = (sg0, sg1)
    ss = (ss0, ss1)
    sd = (sd0, sd1)

    pltpu.sync_copy(as_hbm, as_v)
    pltpu.sync_copy(ad_hbm, ad_v)
    pltpu.sync_copy(m_hbm, m_v)

    zv = jnp.zeros((16,), jnp.float32)

    def _zrow(i, _):
        for c in range(8):
            rows_v[0, i, pl.ds(c * 16, 16)] = zv
        return 0
    lax.fori_loop(0, 128, _zrow, 0)

    apt = _ACC // _NS

    def _zb16(i, _):
        zb_v[pl.ds(i * 16, 16)] = zv
        return 0
    lax.fori_loop(0, apt // 16, _zb16, 0)
    zb_v[pl.ds(apt - 16, 16)] = zv

    row0 = pl.multiple_of(sid * apt, 8)
    pltpu.sync_copy(rows_v.at[0], out_sh.at[pl.ds(row0, 128)])
    pltpu.sync_copy(rows_v.at[0], out_sh.at[pl.ds(row0 + 128, 128)])
    pltpu.sync_copy(rows_v.at[0, pl.ds(0, apt - 256)],
                    out_sh.at[pl.ds(row0 + 256, apt - 256)])
    pltpu.sync_copy(zb_v, denom_sh.at[pl.ds(row0, apt)])

    plsc.subcore_barrier()

    m16 = m_v[...]

    def _load_compute(j, p):
        pltpu.sync_copy(edges_hbm.at[sid, j], inb.at[p])
        for k in range(8):
            si = inb[p, pl.ds(k * 16, 16)]
            di = inb[p, pl.ds(128 + k * 16, 16)]
            a16 = plsc.bitcast(inb[p, pl.ds(256 + k * 16, 16)], jnp.float32)
            sg_ = plsc.load_gather(as_v, [si])
            dg_ = plsc.load_gather(ad_v, [di])
            t = sg_ + dg_ + a16
            exb[p, pl.ds(k * 16, 16)] = jnp.exp(_leaky(t) - m16)
            ld = di - node0
            oor = (ld < 0) | (ld >= _HALF)
            trash = _HALF + (di & 127)
            ldst[p, pl.ds(k * 16, 16)] = jnp.where(oor, trash, ld)

    def _issue_gather(p):
        return pltpu.async_copy(
            h_hbm.at[inb.at[p, pl.ds(0, 128)]], rows_v.at[p], sg[p])

    _load_compute(0, 0)
    _issue_gather(0)

    def _pair(jj, _):
        for p in (0, 1):
            np_ = 1 - p
            j = 2 * jj + p
            jn = j + 1

            @pl.when(j >= 1)
            def _():
                pltpu.make_async_copy(
                    rows_v.at[np_], out_sh.at[ldst.at[np_]], ss[np_]).wait()
                pltpu.make_async_copy(
                    exb.at[np_], denom_sh.at[ldst.at[np_]], sd[np_]).wait()

            @pl.when(jn < _CH)
            def _():
                _load_compute(jn, np_)

            pltpu.make_async_copy(
                h_hbm.at[inb.at[p, pl.ds(0, 128)]], rows_v.at[p],
                sg[p]).wait()
            pltpu.async_copy(
                exb.at[p], denom_sh.at[ldst.at[p]], sd[p], add=True)
            for k in range(8):
                e16 = exb[p, pl.ds(k * 16, 16)]
                for r in range(16):
                    b16 = _bcast_lane(e16, r)
                    row = k * 16 + r
                    for c in range(8):
                        rows_v[p, row, pl.ds(c * 16, 16)] = (
                            rows_v[p, row, pl.ds(c * 16, 16)] * b16)
            pltpu.async_copy(
                rows_v.at[p], out_sh.at[ldst.at[p]], ss[p], add=True)

            @pl.when(jn < _CH)
            def _():
                _issue_gather(np_)
        return 0
    lax.fori_loop(0, _CH // 2, _pair, 0)

    pltpu.make_async_copy(rows_v.at[1], out_sh.at[ldst.at[1]], ss[1]).wait()
    pltpu.make_async_copy(exb.at[1], denom_sh.at[ldst.at[1]], sd[1]).wait()

    plsc.subcore_barrier()

    rpt = _HALF // _NS
    out0 = pl.multiple_of(sid * rpt, 8)
    h0 = pl.multiple_of(cid * _HALF + sid * rpt, 8)
    pltpu.sync_copy(out_sh.at[pl.ds(out0, rpt)], u_hbm.at[pl.ds(h0, rpt)])
    pltpu.sync_copy(denom_sh.at[pl.ds(out0, rpt)], zb_v.at[pl.ds(0, rpt)])
    pltpu.sync_copy(zb_v.at[pl.ds(0, rpt)], d_hbm.at[pl.ds(h0, rpt)])


_BLK = 1024
_GRID = _NP // _BLK


def _store_h_sd_max(h, h_ref, sd_ref, mx_ref, a2_ref):
    sd = jnp.dot(h, a2_ref[...], preferred_element_type=jnp.float32)
    h_ref[...] = h
    sd_ref[...] = sd
    i = pl.program_id(0)
    rid = i * _BLK + lax.broadcasted_iota(jnp.int32, (_BLK, 8), 0)
    mb = jnp.broadcast_to(
        jnp.max(jnp.where(rid < _N, sd, _NEG), axis=0, keepdims=True), (8, 8))

    @pl.when(i == 0)
    def _():
        mx_ref[...] = mb

    @pl.when(i > 0)
    def _():
        mx_ref[...] = jnp.maximum(mx_ref[...], mb)


def _mm_first_body(x_ref, w_ref, a2_ref, h_ref, sd_ref, mx_ref):
    h = jnp.dot(x_ref[...], w_ref[...], preferred_element_type=jnp.float32)
    _store_h_sd_max(h, h_ref, sd_ref, mx_ref, a2_ref)


def _mm_later_body(u_ref, d_ref, bias_ref, w_ref, a2_ref,
                   h_ref, sd_ref, mx_ref):
    den = d_ref[...] + 1e-16
    y = u_ref[...] / den + bias_ref[...]
    y = jnp.maximum(y, 0.0)
    h = jnp.dot(y, w_ref[...], preferred_element_type=jnp.float32)
    _store_h_sd_max(h, h_ref, sd_ref, mx_ref, a2_ref)


_MM_OUT = [
    jax.ShapeDtypeStruct((_NP, 128), jnp.float32),
    jax.ShapeDtypeStruct((_NP, 8), jnp.float32),
    jax.ShapeDtypeStruct((8, 8), jnp.float32),
]
_MM_OUT_SPECS = [
    pl.BlockSpec((_BLK, 128), lambda i: (i, 0)),
    pl.BlockSpec((_BLK, 8), lambda i: (i, 0)),
    pl.BlockSpec((8, 8), lambda i: (0, 0)),
]

_mm_first = pl.pallas_call(
    _mm_first_body,
    grid=(_GRID,),
    in_specs=[
        pl.BlockSpec((_BLK, 128), lambda i: (i, 0)),
        pl.BlockSpec((128, 128), lambda i: (0, 0)),
        pl.BlockSpec((128, 8), lambda i: (0, 0)),
    ],
    out_specs=_MM_OUT_SPECS,
    out_shape=_MM_OUT,
)

_mm_later = pl.pallas_call(
    _mm_later_body,
    grid=(_GRID,),
    in_specs=[
        pl.BlockSpec((_BLK, 128), lambda i: (i, 0)),
        pl.BlockSpec((_BLK, 1), lambda i: (i, 0)),
        pl.BlockSpec((1, 128), lambda i: (0, 0)),
        pl.BlockSpec((128, 128), lambda i: (0, 0)),
        pl.BlockSpec((128, 8), lambda i: (0, 0)),
    ],
    out_specs=_MM_OUT_SPECS,
    out_shape=_MM_OUT,
)


def _ae_body(ea_ref, we_ref, aew_ref, out_ref, mx_ref):
    we = we_ref[...]
    aw = aew_ref[...]
    cols = [jnp.dot(we[l], aw[l], preferred_element_type=jnp.float32)[:, None]
            for l in range(5)]
    cols.append(jnp.zeros((12, 3), jnp.float32))
    v8 = jnp.concatenate(cols, axis=1)
    ae8 = jnp.dot(ea_ref[...], v8, preferred_element_type=jnp.float32)
    out_ref[...] = ae8
    i = pl.program_id(0)
    mb = jnp.broadcast_to(jnp.max(ae8, axis=0, keepdims=True), (8, 8))

    @pl.when(i == 0)
    def _():
        mx_ref[...] = mb

    @pl.when(i > 0)
    def _():
        mx_ref[...] = jnp.maximum(mx_ref[...], mb)


_AE_BLK = 2000
_ae_proj = pl.pallas_call(
    _ae_body,
    grid=(_E // _AE_BLK,),
    in_specs=[
        pl.BlockSpec((_AE_BLK, 12), lambda i: (i, 0)),
        pl.BlockSpec((5, 12, 128), lambda i: (0, 0, 0)),
        pl.BlockSpec((5, 128), lambda i: (0, 0)),
    ],
    out_specs=[
        pl.BlockSpec((_AE_BLK, 8), lambda i: (i, 0)),
        pl.BlockSpec((8, 8), lambda i: (0, 0)),
    ],
    out_shape=[
        jax.ShapeDtypeStruct((_E, 8), jnp.float32),
        jax.ShapeDtypeStruct((8, 8), jnp.float32),
    ],
)


def _pool_body(u_ref, d_ref, bias_ref, bf_ref, wl_ref, bl_ref,
               out_ref, pooled_acc, counts_acc):
    i = pl.program_id(0)

    @pl.when(i == 0)
    def _():
        pooled_acc[...] = jnp.zeros((64, 128), jnp.float32)
        counts_acc[...] = jnp.zeros((64, 128), jnp.float32)

    den = d_ref[...] + 1e-16
    y = u_ref[...] / den + bias_ref[...]
    bf = bf_ref[...]
    g = lax.broadcasted_iota(jnp.int32, (_BLK, 64), 1)
    mask = (bf == g).astype(jnp.float32)
    pooled_acc[...] += lax.dot_general(
        mask, y, (((0,), (0,)), ((), ())), preferred_element_type=jnp.float32)
    counts_acc[...] += jnp.broadcast_to(
        jnp.sum(mask, axis=0)[:, None], (64, 128))

    @pl.when(i == _GRID - 1)
    def _():
        pm = pooled_acc[...] / jnp.maximum(counts_acc[...], 1.0)
        out_ref[...] = jnp.dot(pm, wl_ref[...],
                               preferred_element_type=jnp.float32) + bl_ref[...]


_pool = pl.pallas_call(
    _pool_body,
    grid=(_GRID,),
    in_specs=[
        pl.BlockSpec((_BLK, 128), lambda i: (i, 0)),
        pl.BlockSpec((_BLK, 1), lambda i: (i, 0)),
        pl.BlockSpec((1, 128), lambda i: (0, 0)),
        pl.BlockSpec((_BLK, 1), lambda i: (i, 0)),
        pl.BlockSpec((128, 1), lambda i: (0, 0)),
        pl.BlockSpec((1, 1), lambda i: (0, 0)),
    ],
    out_specs=pl.BlockSpec((64, 1), lambda i: (0, 0)),
    out_shape=jax.ShapeDtypeStruct((64, 1), jnp.float32),
    scratch_shapes=[
        pltpu.VMEM((64, 128), jnp.float32),
        pltpu.VMEM((64, 128), jnp.float32),
    ],
)


def _tile_layout(a, pad_val):
    a2 = a.reshape(_NS, _E // _NS)
    pad = jnp.full((_NS, _EPT - _E // _NS), pad_val, a.dtype)
    return jnp.concatenate([a2, pad], axis=1).reshape(_NS, _CH, 128)


def kernel(x, edge_index, edge_attr, batch, W, a_src, a_dst, W_e, a_e, b,
           W_lin, b_lin):
    f32 = jnp.float32
    src_t = _tile_layout(edge_index[0], 0)
    dst_t = _tile_layout(edge_index[1], 0)

    ae8, mxae = _ae_proj(edge_attr, W_e, a_e)

    x_pad = jnp.concatenate([x, jnp.zeros((_NP - _N, 128), f32)], axis=0)
    batchf = jnp.concatenate(
        [batch, jnp.full((_NP - _N,), 64, jnp.int32)])[:, None]

    u = d = None
    for l in range(5):
        a2 = jnp.concatenate(
            [a_src[l][:, None], a_dst[l][:, None], jnp.zeros((128, 6), f32)],
            axis=1)
        if l == 0:
            h, sd, mxsd = _mm_first(x_pad, W[0], a2)
        else:
            h, sd, mxsd = _mm_later(u, d[:, None], b[l - 1][None, :],
                                    W[l], a2)
        as_l = sd[:_N, 0]
        ad_l = sd[:_N, 1]
        m_val = _leaky(mxsd[0, 0] + mxsd[0, 1] + mxae[0, l])
        m_arr = jnp.full((16,), m_val, f32)
        ae_l = _tile_layout(ae8[:, l], _NEG)
        edges_l = jnp.concatenate(
            [src_t, dst_t, lax.bitcast_convert_type(ae_l, jnp.int32)], axis=2)
        u, d = _sc_edge(edges_l, as_l, ad_l, h, m_arr)

    return _pool(u, d[:, None], b[4][None, :], batchf, W_lin,
                 b_lin[:, None])


# speedup vs baseline: 14.4822x; 1.2431x over previous
import functools
import jax
import jax.numpy as jnp
from jax import lax
from jax.experimental import pallas as pl
from jax.experimental.pallas import tpu as pltpu
from jax.experimental.pallas import tpu_sc as plsc

_NS = 16
_N = 10000
_NP = 10240
_E = 320000
_CH = 158
_EPT = _CH * 128
_NEG = -1e30
_HALF = _NP // 2
_ACC = _HALF + 128


def _leaky(t):
    return jnp.maximum(t, 0.2 * t)


def _bcast_lane(v, r):
    idx = jnp.full((16, 1), r, dtype=jnp.int32)
    dn = lax.GatherDimensionNumbers(
        offset_dims=(), collapsed_slice_dims=(0,), start_index_map=(0,))
    return lax.gather(v, idx, dn, (1,),
                      mode=lax.GatherScatterMode.PROMISE_IN_BOUNDS)


@functools.partial(
    pl.kernel,
    out_type=[
        jax.ShapeDtypeStruct((_NP, 128), jnp.float32),
        jax.ShapeDtypeStruct((_NP,), jnp.float32),
    ],
    mesh=plsc.VectorSubcoreMesh(core_axis_name="c", subcore_axis_name="s"),
    compiler_params=pltpu.CompilerParams(needs_layout_passes=False),
    scratch_types=[
        pltpu.VMEM((2, 384), jnp.int32),
        pltpu.VMEM((2, 128), jnp.int32),
        pltpu.VMEM((2, 128), jnp.float32),
        pltpu.VMEM((_N,), jnp.float32),
        pltpu.VMEM((_N,), jnp.float32),
        pltpu.VMEM((2, 128, 128), jnp.float32),
        pltpu.VMEM((_ACC // _NS,), jnp.float32),
        pltpu.VMEM((16,), jnp.float32),
        pltpu.SemaphoreType.DMA,
        pltpu.SemaphoreType.DMA,
        pltpu.SemaphoreType.DMA,
        pltpu.SemaphoreType.DMA,
        pltpu.SemaphoreType.DMA,
        pltpu.SemaphoreType.DMA,
        pltpu.SMEM((2,), jnp.int32),
        pltpu.VMEM_SHARED((_ACC,), jnp.float32),
        pltpu.VMEM_SHARED((_ACC, 128), jnp.float32),
    ],
)
def _sc_edge(edges_hbm, as_hbm, ad_hbm, h_hbm, m_hbm,
             u_hbm, d_hbm,
             inb, ldst, exb, as_v, ad_v, rows_v, zb_v, m_v,
             sg0, sg1, ss0, ss1, sd0, sd1, rel_s,
             denom_sh, out_sh):
    cid = lax.axis_index("c")
    sid = lax.axis_index("s")
    node0 = cid * _HALF
    sg = (sg0, sg1)
    ss = (ss0, ss1)
    sd = (sd0, sd1)

    pltpu.sync_copy(as_hbm, as_v)
    pltpu.sync_copy(ad_hbm, ad_v)
    pltpu.sync_copy(m_hbm, m_v)

    zv = jnp.zeros((16,), jnp.float32)

    def _zrow(i, _):
        for c in range(8):
            rows_v[0, i, pl.ds(c * 16, 16)] = zv
        return 0
    lax.fori_loop(0, 128, _zrow, 0)

    apt = _ACC // _NS

    def _zb16(i, _):
        zb_v[pl.ds(i * 16, 16)] = zv
        return 0
    lax.fori_loop(0, apt // 16, _zb16, 0)
    zb_v[pl.ds(apt - 16, 16)] = zv

    row0 = pl.multiple_of(sid * apt, 8)
    pltpu.sync_copy(rows_v.at[0], out_sh.at[pl.ds(row0, 128)])
    pltpu.sync_copy(rows_v.at[0], out_sh.at[pl.ds(row0 + 128, 128)])
    pltpu.sync_copy(rows_v.at[0, pl.ds(0, apt - 256)],
                    out_sh.at[pl.ds(row0 + 256, apt - 256)])
    pltpu.sync_copy(zb_v, denom_sh.at[pl.ds(row0, apt)])

    plsc.subcore_barrier()

    m16 = m_v[...]

    def _load_compute(j, p):
        pltpu.sync_copy(edges_hbm.at[sid, j], inb.at[p])
        rel16 = jnp.zeros((16,), jnp.int32)
        for k in range(8):
            si = inb[p, pl.ds(k * 16, 16)]
            di = inb[p, pl.ds(128 + k * 16, 16)]
            a16 = plsc.bitcast(inb[p, pl.ds(256 + k * 16, 16)], jnp.float32)
            sg_ = plsc.load_gather(as_v, [si])
            dg_ = plsc.load_gather(ad_v, [di])
            t = sg_ + dg_ + a16
            exb[p, pl.ds(k * 16, 16)] = jnp.exp(_leaky(t) - m16)
            ld = di - node0
            oor = (ld < 0) | (ld >= _HALF)
            trash = _HALF + (di & 127)
            ldst[p, pl.ds(k * 16, 16)] = jnp.where(oor, trash, ld)
            rel16 = rel16 | jnp.where(oor, 0, 1)
        rel_s[p] = jnp.max(rel16)

    def _issue_gather(p):
        return pltpu.async_copy(
            h_hbm.at[inb.at[p, pl.ds(0, 128)]], rows_v.at[p], sg[p])

    _load_compute(0, 0)

    @pl.when(rel_s[0] > 0)
    def _():
        _issue_gather(0)

    def _pair(jj, _):
        for p in (0, 1):
            np_ = 1 - p
            j = 2 * jj + p
            jn = j + 1

            @pl.when((j >= 1) & (rel_s[np_] > 0))
            def _():
                pltpu.make_async_copy(
                    rows_v.at[np_], out_sh.at[ldst.at[np_]], ss[np_]).wait()
                pltpu.make_async_copy(
                    exb.at[np_], denom_sh.at[ldst.at[np_]], sd[np_]).wait()

            @pl.when(jn < _CH)
            def _():
                _load_compute(jn, np_)

            @pl.when(rel_s[p] > 0)
            def _():
                pltpu.make_async_copy(
                    h_hbm.at[inb.at[p, pl.ds(0, 128)]], rows_v.at[p],
                    sg[p]).wait()
                pltpu.async_copy(
                    exb.at[p], denom_sh.at[ldst.at[p]], sd[p], add=True)
                for k in range(8):
                    e16 = exb[p, pl.ds(k * 16, 16)]
                    for r in range(16):
                        b16 = _bcast_lane(e16, r)
                        row = k * 16 + r
                        for c in range(8):
                            rows_v[p, row, pl.ds(c * 16, 16)] = (
                                rows_v[p, row, pl.ds(c * 16, 16)] * b16)
                pltpu.async_copy(
                    rows_v.at[p], out_sh.at[ldst.at[p]], ss[p], add=True)

            @pl.when((jn < _CH) & (rel_s[np_] > 0))
            def _():
                _issue_gather(np_)
        return 0
    lax.fori_loop(0, _CH // 2, _pair, 0)

    @pl.when(rel_s[1] > 0)
    def _():
        pltpu.make_async_copy(
            rows_v.at[1], out_sh.at[ldst.at[1]], ss[1]).wait()
        pltpu.make_async_copy(
            exb.at[1], denom_sh.at[ldst.at[1]], sd[1]).wait()

    plsc.subcore_barrier()

    rpt = _HALF // _NS
    out0 = pl.multiple_of(sid * rpt, 8)
    h0 = pl.multiple_of(cid * _HALF + sid * rpt, 8)
    pltpu.sync_copy(out_sh.at[pl.ds(out0, rpt)], u_hbm.at[pl.ds(h0, rpt)])
    pltpu.sync_copy(denom_sh.at[pl.ds(out0, rpt)], zb_v.at[pl.ds(0, rpt)])
    pltpu.sync_copy(zb_v.at[pl.ds(0, rpt)], d_hbm.at[pl.ds(h0, rpt)])


_BLK = 1024
_GRID = _NP // _BLK


def _store_h_sd_max(h, h_ref, sd_ref, mx_ref, a2_ref):
    sd = jnp.dot(h, a2_ref[...], preferred_element_type=jnp.float32)
    h_ref[...] = h
    sd_ref[...] = sd
    i = pl.program_id(0)
    rid = i * _BLK + lax.broadcasted_iota(jnp.int32, (_BLK, 8), 0)
    mb = jnp.broadcast_to(
        jnp.max(jnp.where(rid < _N, sd, _NEG), axis=0, keepdims=True), (8, 8))

    @pl.when(i == 0)
    def _():
        mx_ref[...] = mb

    @pl.when(i > 0)
    def _():
        mx_ref[...] = jnp.maximum(mx_ref[...], mb)


def _mm_first_body(x_ref, w_ref, a2_ref, h_ref, sd_ref, mx_ref):
    h = jnp.dot(x_ref[...], w_ref[...], preferred_element_type=jnp.float32)
    _store_h_sd_max(h, h_ref, sd_ref, mx_ref, a2_ref)


def _mm_later_body(u_ref, d_ref, bias_ref, w_ref, a2_ref,
                   h_ref, sd_ref, mx_ref):
    den = d_ref[...] + 1e-16
    y = u_ref[...] / den + bias_ref[...]
    y = jnp.maximum(y, 0.0)
    h = jnp.dot(y, w_ref[...], preferred_element_type=jnp.float32)
    _store_h_sd_max(h, h_ref, sd_ref, mx_ref, a2_ref)


_MM_OUT = [
    jax.ShapeDtypeStruct((_NP, 128), jnp.float32),
    jax.ShapeDtypeStruct((_NP, 8), jnp.float32),
    jax.ShapeDtypeStruct((8, 8), jnp.float32),
]
_MM_OUT_SPECS = [
    pl.BlockSpec((_BLK, 128), lambda i: (i, 0)),
    pl.BlockSpec((_BLK, 8), lambda i: (i, 0)),
    pl.BlockSpec((8, 8), lambda i: (0, 0)),
]

_mm_first = pl.pallas_call(
    _mm_first_body,
    grid=(_GRID,),
    in_specs=[
        pl.BlockSpec((_BLK, 128), lambda i: (i, 0)),
        pl.BlockSpec((128, 128), lambda i: (0, 0)),
        pl.BlockSpec((128, 8), lambda i: (0, 0)),
    ],
    out_specs=_MM_OUT_SPECS,
    out_shape=_MM_OUT,
)

_mm_later = pl.pallas_call(
    _mm_later_body,
    grid=(_GRID,),
    in_specs=[
        pl.BlockSpec((_BLK, 128), lambda i: (i, 0)),
        pl.BlockSpec((_BLK, 1), lambda i: (i, 0)),
        pl.BlockSpec((1, 128), lambda i: (0, 0)),
        pl.BlockSpec((128, 128), lambda i: (0, 0)),
        pl.BlockSpec((128, 8), lambda i: (0, 0)),
    ],
    out_specs=_MM_OUT_SPECS,
    out_shape=_MM_OUT,
)


def _ae_body(ea_ref, we_ref, aew_ref, out_ref, mx_ref):
    we = we_ref[...]
    aw = aew_ref[...]
    cols = [jnp.dot(we[l], aw[l], preferred_element_type=jnp.float32)[:, None]
            for l in range(5)]
    cols.append(jnp.zeros((12, 3), jnp.float32))
    v8 = jnp.concatenate(cols, axis=1)
    ae8 = jnp.dot(ea_ref[...], v8, preferred_element_type=jnp.float32)
    out_ref[...] = ae8
    i = pl.program_id(0)
    mb = jnp.broadcast_to(jnp.max(ae8, axis=0, keepdims=True), (8, 8))

    @pl.when(i == 0)
    def _():
        mx_ref[...] = mb

    @pl.when(i > 0)
    def _():
        mx_ref[...] = jnp.maximum(mx_ref[...], mb)


_AE_BLK = 2000
_ae_proj = pl.pallas_call(
    _ae_body,
    grid=(_E // _AE_BLK,),
    in_specs=[
        pl.BlockSpec((_AE_BLK, 12), lambda i: (i, 0)),
        pl.BlockSpec((5, 12, 128), lambda i: (0, 0, 0)),
        pl.BlockSpec((5, 128), lambda i: (0, 0)),
    ],
    out_specs=[
        pl.BlockSpec((_AE_BLK, 8), lambda i: (i, 0)),
        pl.BlockSpec((8, 8), lambda i: (0, 0)),
    ],
    out_shape=[
        jax.ShapeDtypeStruct((_E, 8), jnp.float32),
        jax.ShapeDtypeStruct((8, 8), jnp.float32),
    ],
)


def _pool_body(u_ref, d_ref, bias_ref, bf_ref, wl_ref, bl_ref,
               out_ref, pooled_acc, counts_acc):
    i = pl.program_id(0)

    @pl.when(i == 0)
    def _():
        pooled_acc[...] = jnp.zeros((64, 128), jnp.float32)
        counts_acc[...] = jnp.zeros((64, 128), jnp.float32)

    den = d_ref[...] + 1e-16
    y = u_ref[...] / den + bias_ref[...]
    bf = bf_ref[...]
    g = lax.broadcasted_iota(jnp.int32, (_BLK, 64), 1)
    mask = (bf == g).astype(jnp.float32)
    pooled_acc[...] += lax.dot_general(
        mask, y, (((0,), (0,)), ((), ())), preferred_element_type=jnp.float32)
    counts_acc[...] += jnp.broadcast_to(
        jnp.sum(mask, axis=0)[:, None], (64, 128))

    @pl.when(i == _GRID - 1)
    def _():
        pm = pooled_acc[...] / jnp.maximum(counts_acc[...], 1.0)
        out_ref[...] = jnp.dot(pm, wl_ref[...],
                               preferred_element_type=jnp.float32) + bl_ref[...]


_pool = pl.pallas_call(
    _pool_body,
    grid=(_GRID,),
    in_specs=[
        pl.BlockSpec((_BLK, 128), lambda i: (i, 0)),
        pl.BlockSpec((_BLK, 1), lambda i: (i, 0)),
        pl.BlockSpec((1, 128), lambda i: (0, 0)),
        pl.BlockSpec((_BLK, 1), lambda i: (i, 0)),
        pl.BlockSpec((128, 1), lambda i: (0, 0)),
        pl.BlockSpec((1, 1), lambda i: (0, 0)),
    ],
    out_specs=pl.BlockSpec((64, 1), lambda i: (0, 0)),
    out_shape=jax.ShapeDtypeStruct((64, 1), jnp.float32),
    scratch_shapes=[
        pltpu.VMEM((64, 128), jnp.float32),
        pltpu.VMEM((64, 128), jnp.float32),
    ],
)


def _tile_layout(a, pad_val):
    a2 = a.reshape(_NS, _E // _NS)
    pad = jnp.full((_NS, _EPT - _E // _NS), pad_val, a.dtype)
    return jnp.concatenate([a2, pad], axis=1).reshape(_NS, _CH, 128)


def kernel(x, edge_index, edge_attr, batch, W, a_src, a_dst, W_e, a_e, b,
           W_lin, b_lin):
    f32 = jnp.float32
    ae8, mxae = _ae_proj(edge_attr, W_e, a_e)

    perm = jnp.argsort(edge_index[1])
    src_t = _tile_layout(edge_index[0][perm], 0)
    dst_t = _tile_layout(edge_index[1][perm], 0)
    ae8 = ae8[perm]

    x_pad = jnp.concatenate([x, jnp.zeros((_NP - _N, 128), f32)], axis=0)
    batchf = jnp.concatenate(
        [batch, jnp.full((_NP - _N,), 64, jnp.int32)])[:, None]

    u = d = None
    for l in range(5):
        a2 = jnp.concatenate(
            [a_src[l][:, None], a_dst[l][:, None], jnp.zeros((128, 6), f32)],
            axis=1)
        if l == 0:
            h, sd, mxsd = _mm_first(x_pad, W[0], a2)
        else:
            h, sd, mxsd = _mm_later(u, d[:, None], b[l - 1][None, :],
                                    W[l], a2)
        as_l = sd[:_N, 0]
        ad_l = sd[:_N, 1]
        m_val = _leaky(mxsd[0, 0] + mxsd[0, 1] + mxae[0, l])
        m_arr = jnp.full((16,), m_val, f32)
        ae_l = _tile_layout(ae8[:, l], _NEG)
        edges_l = jnp.concatenate(
            [src_t, dst_t, lax.bitcast_convert_type(ae_l, jnp.int32)], axis=2)
        u, d = _sc_edge(edges_l, as_l, ad_l, h, m_arr)

    return _pool(u, d[:, None], b[4][None, :], batchf, W_lin,
                 b_lin[:, None])


# speedup vs baseline: 14.7016x; 1.0152x over previous
import functools
import jax
import jax.numpy as jnp
from jax import lax
from jax.experimental import pallas as pl
from jax.experimental.pallas import tpu as pltpu
from jax.experimental.pallas import tpu_sc as plsc

_NS = 16
_N = 10000
_NP = 10240
_E = 320000
_CH = 158
_EPT = _CH * 128
_NEG = -1e30
_HALF = _NP // 2
_ACC = _HALF + 128


def _leaky(t):
    return jnp.maximum(t, 0.2 * t)


def _bcast_lane(v, r):
    idx = jnp.full((16, 1), r, dtype=jnp.int32)
    dn = lax.GatherDimensionNumbers(
        offset_dims=(), collapsed_slice_dims=(0,), start_index_map=(0,))
    return lax.gather(v, idx, dn, (1,),
                      mode=lax.GatherScatterMode.PROMISE_IN_BOUNDS)


@functools.partial(
    pl.kernel,
    out_type=[
        jax.ShapeDtypeStruct((_NP, 128), jnp.float32),
        jax.ShapeDtypeStruct((_NP,), jnp.float32),
    ],
    mesh=plsc.VectorSubcoreMesh(core_axis_name="c", subcore_axis_name="s"),
    compiler_params=pltpu.CompilerParams(needs_layout_passes=False),
    scratch_types=[
        pltpu.VMEM((2, 384), jnp.int32),
        pltpu.VMEM((2, 128), jnp.int32),
        pltpu.VMEM((2, 128), jnp.float32),
        pltpu.VMEM((_N,), jnp.float32),
        pltpu.VMEM((_N,), jnp.float32),
        pltpu.VMEM((2, 128, 128), jnp.float32),
        pltpu.VMEM((_ACC // _NS,), jnp.float32),
        pltpu.VMEM((16,), jnp.float32),
        pltpu.SemaphoreType.DMA,
        pltpu.SemaphoreType.DMA,
        pltpu.SemaphoreType.DMA,
        pltpu.SemaphoreType.DMA,
        pltpu.SemaphoreType.DMA,
        pltpu.SemaphoreType.DMA,
        pltpu.SMEM((2,), jnp.int32),
        pltpu.VMEM_SHARED((_ACC,), jnp.float32),
        pltpu.VMEM_SHARED((_ACC, 128), jnp.float32),
    ],
)
def _sc_edge(edges_hbm, as_hbm, ad_hbm, h_hbm, m_hbm,
             u_hbm, d_hbm,
             inb, ldst, exb, as_v, ad_v, rows_v, zb_v, m_v,
             sg0, sg1, ss0, ss1, sd0, sd1, rel_s,
             denom_sh, out_sh):
    cid = lax.axis_index("c")
    sid = lax.axis_index("s")
    node0 = cid * _HALF
    sg = (sg0, sg1)
    ss = (ss0, ss1)
    sd = (sd0, sd1)

    pltpu.sync_copy(as_hbm, as_v)
    pltpu.sync_copy(ad_hbm, ad_v)
    pltpu.sync_copy(m_hbm, m_v)

    zv = jnp.zeros((16,), jnp.float32)

    def _zrow(i, _):
        for c in range(8):
            rows_v[0, i, pl.ds(c * 16, 16)] = zv
        return 0
    lax.fori_loop(0, 128, _zrow, 0)

    apt = _ACC // _NS

    def _zb16(i, _):
        zb_v[pl.ds(i * 16, 16)] = zv
        return 0
    lax.fori_loop(0, apt // 16, _zb16, 0)
    zb_v[pl.ds(apt - 16, 16)] = zv

    row0 = pl.multiple_of(sid * apt, 8)
    pltpu.sync_copy(rows_v.at[0], out_sh.at[pl.ds(row0, 128)])
    pltpu.sync_copy(rows_v.at[0], out_sh.at[pl.ds(row0 + 128, 128)])
    pltpu.sync_copy(rows_v.at[0, pl.ds(0, apt - 256)],
                    out_sh.at[pl.ds(row0 + 256, apt - 256)])
    pltpu.sync_copy(zb_v, denom_sh.at[pl.ds(row0, apt)])

    plsc.subcore_barrier()

    m16 = m_v[...]

    def _load_compute(j, p):
        pltpu.sync_copy(edges_hbm.at[sid, j], inb.at[p])
        rel16 = jnp.zeros((16,), jnp.int32)
        for k in range(8):
            si = inb[p, pl.ds(k * 16, 16)]
            di = inb[p, pl.ds(128 + k * 16, 16)]
            a16 = plsc.bitcast(inb[p, pl.ds(256 + k * 16, 16)], jnp.float32)
            sg_ = plsc.load_gather(as_v, [si])
            dg_ = plsc.load_gather(ad_v, [di])
            t = sg_ + dg_ + a16
            exb[p, pl.ds(k * 16, 16)] = jnp.exp(_leaky(t) - m16)
            ld = di - node0
            oor = (ld < 0) | (ld >= _HALF)
            trash = _HALF + (di & 127)
            ldst[p, pl.ds(k * 16, 16)] = jnp.where(oor, trash, ld)
            rel16 = rel16 | jnp.where(oor, 0, 1)
        rel_s[p] = jnp.max(rel16)

    def _issue_gather(p):
        return pltpu.async_copy(
            h_hbm.at[inb.at[p, pl.ds(0, 128)]], rows_v.at[p], sg[p])

    _load_compute(0, 0)

    @pl.when(rel_s[0] > 0)
    def _():
        _issue_gather(0)

    def _pair(jj, _):
        for p in (0, 1):
            np_ = 1 - p
            j = 2 * jj + p
            jn = j + 1

            @pl.when((j >= 1) & (rel_s[np_] > 0))
            def _():
                pltpu.make_async_copy(
                    rows_v.at[np_], out_sh.at[ldst.at[np_]], ss[np_]).wait()
                pltpu.make_async_copy(
                    exb.at[np_], denom_sh.at[ldst.at[np_]], sd[np_]).wait()

            @pl.when(jn < _CH)
            def _():
                _load_compute(jn, np_)

            @pl.when(rel_s[p] > 0)
            def _():
                pltpu.make_async_copy(
                    h_hbm.at[inb.at[p, pl.ds(0, 128)]], rows_v.at[p],
                    sg[p]).wait()
                pltpu.async_copy(
                    exb.at[p], denom_sh.at[ldst.at[p]], sd[p], add=True)
                for k in range(8):
                    e16 = exb[p, pl.ds(k * 16, 16)]
                    for r in range(16):
                        b16 = _bcast_lane(e16, r)
                        row = k * 16 + r
                        for c in range(8):
                            rows_v[p, row, pl.ds(c * 16, 16)] = (
                                rows_v[p, row, pl.ds(c * 16, 16)] * b16)
                pltpu.async_copy(
                    rows_v.at[p], out_sh.at[ldst.at[p]], ss[p], add=True)

            @pl.when((jn < _CH) & (rel_s[np_] > 0))
            def _():
                _issue_gather(np_)
        return 0
    lax.fori_loop(0, _CH // 2, _pair, 0)

    @pl.when(rel_s[1] > 0)
    def _():
        pltpu.make_async_copy(
            rows_v.at[1], out_sh.at[ldst.at[1]], ss[1]).wait()
        pltpu.make_async_copy(
            exb.at[1], denom_sh.at[ldst.at[1]], sd[1]).wait()

    plsc.subcore_barrier()

    rpt = _HALF // _NS
    out0 = pl.multiple_of(sid * rpt, 8)
    h0 = pl.multiple_of(cid * _HALF + sid * rpt, 8)
    pltpu.sync_copy(out_sh.at[pl.ds(out0, rpt)], u_hbm.at[pl.ds(h0, rpt)])
    pltpu.sync_copy(denom_sh.at[pl.ds(out0, rpt)], zb_v.at[pl.ds(0, rpt)])
    pltpu.sync_copy(zb_v.at[pl.ds(0, rpt)], d_hbm.at[pl.ds(h0, rpt)])


_BLK = 1024
_GRID = _NP // _BLK


def _store_h_sd_max(h, h_ref, sd_ref, mx_ref, a2_ref):
    sd = jnp.dot(h, a2_ref[...], preferred_element_type=jnp.float32)
    h_ref[...] = h
    sd_ref[...] = sd
    i = pl.program_id(0)
    rid = i * _BLK + lax.broadcasted_iota(jnp.int32, (_BLK, 8), 0)
    mb = jnp.broadcast_to(
        jnp.max(jnp.where(rid < _N, sd, _NEG), axis=0, keepdims=True), (8, 8))

    @pl.when(i == 0)
    def _():
        mx_ref[...] = mb

    @pl.when(i > 0)
    def _():
        mx_ref[...] = jnp.maximum(mx_ref[...], mb)


def _mm_first_body(x_ref, w_ref, a2_ref, h_ref, sd_ref, mx_ref):
    h = jnp.dot(x_ref[...], w_ref[...], preferred_element_type=jnp.float32)
    _store_h_sd_max(h, h_ref, sd_ref, mx_ref, a2_ref)


def _mm_later_body(u_ref, d_ref, bias_ref, w_ref, a2_ref,
                   h_ref, sd_ref, mx_ref):
    den = d_ref[...] + 1e-16
    y = u_ref[...] / den + bias_ref[...]
    y = jnp.maximum(y, 0.0)
    h = jnp.dot(y, w_ref[...], preferred_element_type=jnp.float32)
    _store_h_sd_max(h, h_ref, sd_ref, mx_ref, a2_ref)


_MM_OUT = [
    jax.ShapeDtypeStruct((_NP, 128), jnp.float32),
    jax.ShapeDtypeStruct((_NP, 8), jnp.float32),
    jax.ShapeDtypeStruct((8, 8), jnp.float32),
]
_MM_OUT_SPECS = [
    pl.BlockSpec((_BLK, 128), lambda i: (i, 0)),
    pl.BlockSpec((_BLK, 8), lambda i: (i, 0)),
    pl.BlockSpec((8, 8), lambda i: (0, 0)),
]

_mm_first = pl.pallas_call(
    _mm_first_body,
    grid=(_GRID,),
    in_specs=[
        pl.BlockSpec((_BLK, 128), lambda i: (i, 0)),
        pl.BlockSpec((128, 128), lambda i: (0, 0)),
        pl.BlockSpec((128, 8), lambda i: (0, 0)),
    ],
    out_specs=_MM_OUT_SPECS,
    out_shape=_MM_OUT,
)

_mm_later = pl.pallas_call(
    _mm_later_body,
    grid=(_GRID,),
    in_specs=[
        pl.BlockSpec((_BLK, 128), lambda i: (i, 0)),
        pl.BlockSpec((_BLK, 1), lambda i: (i, 0)),
        pl.BlockSpec((1, 128), lambda i: (0, 0)),
        pl.BlockSpec((128, 128), lambda i: (0, 0)),
        pl.BlockSpec((128, 8), lambda i: (0, 0)),
    ],
    out_specs=_MM_OUT_SPECS,
    out_shape=_MM_OUT,
)


def _ae_body(ea_ref, we_ref, aew_ref, out_ref, mx_ref):
    we = we_ref[...]
    aw = aew_ref[...]
    cols = [jnp.dot(we[l], aw[l], preferred_element_type=jnp.float32)[:, None]
            for l in range(5)]
    cols.append(jnp.zeros((12, 3), jnp.float32))
    v8 = jnp.concatenate(cols, axis=1)
    ae8 = jnp.dot(ea_ref[...], v8, preferred_element_type=jnp.float32)
    out_ref[...] = ae8
    i = pl.program_id(0)
    mb = jnp.broadcast_to(jnp.max(ae8, axis=0, keepdims=True), (8, 8))

    @pl.when(i == 0)
    def _():
        mx_ref[...] = mb

    @pl.when(i > 0)
    def _():
        mx_ref[...] = jnp.maximum(mx_ref[...], mb)


_AE_BLK = 2000
_ae_proj = pl.pallas_call(
    _ae_body,
    grid=(_E // _AE_BLK,),
    in_specs=[
        pl.BlockSpec((_AE_BLK, 12), lambda i: (i, 0)),
        pl.BlockSpec((5, 12, 128), lambda i: (0, 0, 0)),
        pl.BlockSpec((5, 128), lambda i: (0, 0)),
    ],
    out_specs=[
        pl.BlockSpec((_AE_BLK, 8), lambda i: (i, 0)),
        pl.BlockSpec((8, 8), lambda i: (0, 0)),
    ],
    out_shape=[
        jax.ShapeDtypeStruct((_E, 8), jnp.float32),
        jax.ShapeDtypeStruct((8, 8), jnp.float32),
    ],
)


def _pool_body(u_ref, d_ref, bias_ref, bf_ref, wl_ref, bl_ref,
               out_ref, pooled_acc, counts_acc):
    i = pl.program_id(0)

    @pl.when(i == 0)
    def _():
        pooled_acc[...] = jnp.zeros((64, 128), jnp.float32)
        counts_acc[...] = jnp.zeros((64, 128), jnp.float32)

    den = d_ref[...] + 1e-16
    y = u_ref[...] / den + bias_ref[...]
    bf = bf_ref[...]
    g = lax.broadcasted_iota(jnp.int32, (_BLK, 64), 1)
    mask = (bf == g).astype(jnp.float32)
    pooled_acc[...] += lax.dot_general(
        mask, y, (((0,), (0,)), ((), ())), preferred_element_type=jnp.float32)
    counts_acc[...] += jnp.broadcast_to(
        jnp.sum(mask, axis=0)[:, None], (64, 128))

    @pl.when(i == _GRID - 1)
    def _():
        pm = pooled_acc[...] / jnp.maximum(counts_acc[...], 1.0)
        out_ref[...] = jnp.dot(pm, wl_ref[...],
                               preferred_element_type=jnp.float32) + bl_ref[...]


_pool = pl.pallas_call(
    _pool_body,
    grid=(_GRID,),
    in_specs=[
        pl.BlockSpec((_BLK, 128), lambda i: (i, 0)),
        pl.BlockSpec((_BLK, 1), lambda i: (i, 0)),
        pl.BlockSpec((1, 128), lambda i: (0, 0)),
        pl.BlockSpec((_BLK, 1), lambda i: (i, 0)),
        pl.BlockSpec((128, 1), lambda i: (0, 0)),
        pl.BlockSpec((1, 1), lambda i: (0, 0)),
    ],
    out_specs=pl.BlockSpec((64, 1), lambda i: (0, 0)),
    out_shape=jax.ShapeDtypeStruct((64, 1), jnp.float32),
    scratch_shapes=[
        pltpu.VMEM((64, 128), jnp.float32),
        pltpu.VMEM((64, 128), jnp.float32),
    ],
)


def _tile_layout(a, pad_val):
    lo = a[:_E // 2].reshape(_NS, _E // 32)
    hi = a[_E // 2:].reshape(_NS, _E // 32)
    pad = jnp.full((_NS, _EPT - _E // _NS), pad_val, a.dtype)
    return jnp.concatenate([lo, hi, pad], axis=1).reshape(_NS, _CH, 128)


def kernel(x, edge_index, edge_attr, batch, W, a_src, a_dst, W_e, a_e, b,
           W_lin, b_lin):
    f32 = jnp.float32
    ae8, mxae = _ae_proj(edge_attr, W_e, a_e)

    perm = jnp.argsort(edge_index[1])
    src_t = _tile_layout(edge_index[0][perm], 0)
    dst_t = _tile_layout(edge_index[1][perm], 0)
    ae8 = ae8[perm]

    x_pad = jnp.concatenate([x, jnp.zeros((_NP - _N, 128), f32)], axis=0)
    batchf = jnp.concatenate(
        [batch, jnp.full((_NP - _N,), 64, jnp.int32)])[:, None]

    u = d = None
    for l in range(5):
        a2 = jnp.concatenate(
            [a_src[l][:, None], a_dst[l][:, None], jnp.zeros((128, 6), f32)],
            axis=1)
        if l == 0:
            h, sd, mxsd = _mm_first(x_pad, W[0], a2)
        else:
            h, sd, mxsd = _mm_later(u, d[:, None], b[l - 1][None, :],
                                    W[l], a2)
        as_l = sd[:_N, 0]
        ad_l = sd[:_N, 1]
        m_val = _leaky(mxsd[0, 0] + mxsd[0, 1] + mxae[0, l])
        m_arr = jnp.full((16,), m_val, f32)
        ae_l = _tile_layout(ae8[:, l], _NEG)
        edges_l = jnp.concatenate(
            [src_t, dst_t, lax.bitcast_convert_type(ae_l, jnp.int32)], axis=2)
        u, d = _sc_edge(edges_l, as_l, ad_l, h, m_arr)

    return _pool(u, d[:, None], b[4][None, :], batchf, W_lin,
                 b_lin[:, None])
